# Initial kernel scaffold; baseline (speedup 1.0000x reference)
import jax
import jax.numpy as jnp
from jax.experimental import pallas as pl


def kernel(X, C, W_node, b_node, W_edge, b_edge, Wm1, bm1, Wm2, bm2, Wd1, bd1, Wd2, bd2, We1, be1, We2, be2):
    raise NotImplementedError("write your pallas kernel here")


# speedup vs baseline: 7.5849x; 7.5849x over previous
import functools

import jax
import jax.numpy as jnp
from jax.experimental import pallas as pl
from jax.experimental.pallas import tpu as pltpu
from jax.experimental.pallas import tpu_sc as plsc

N = 10000
K = 30
DH = 128
NUM_RBF = 16
E = N * K
EPAD = 307200
RB = 200
EB = RB * K
NBLK = N // RB


def _softplus(x):
    return jnp.maximum(x, 0.0) + jnp.log1p(jnp.exp(-jnp.abs(x)))


def _ln(x):
    mu = jnp.mean(x, axis=-1, keepdims=True)
    var = jnp.mean((x - mu) ** 2, axis=-1, keepdims=True)
    return (x - mu) / jnp.sqrt(var + 1e-5)


def _knn_embed_body(x2_ref, x2t_ref, wn_ref, bn_ref,
                    xc_ref, d2sel_ref, eidx_ref, nh_ref, d2_ref):
    i = pl.program_id(0)
    x2 = x2_ref[...]
    x2t = x2t_ref[...]

    xr = (x2[:, 0:1] + x2[:, 3:4] + x2[:, 6:7] + x2[:, 9:10]) * 0.25
    yr = (x2[:, 1:2] + x2[:, 4:5] + x2[:, 7:8] + x2[:, 10:11]) * 0.25
    zr = (x2[:, 2:3] + x2[:, 5:6] + x2[:, 8:9] + x2[:, 11:12]) * 0.25
    xT = (x2t[0:1] + x2t[3:4] + x2t[6:7] + x2t[9:10]) * 0.25
    yT = (x2t[1:2] + x2t[4:5] + x2t[7:8] + x2t[10:11]) * 0.25
    zT = (x2t[2:3] + x2t[5:6] + x2t[8:9] + x2t[11:12]) * 0.25

    dx = xr - xT
    dy = yr - yT
    dz = zr - zT
    d2 = dx * dx + dy * dy + dz * dz
    row_ids = i * RB + jax.lax.broadcasted_iota(jnp.int32, (RB, 1), 0)
    col_ids = jax.lax.broadcasted_iota(jnp.int32, (1, N), 1)
    d2 = jnp.where(row_ids == col_ids, 1e9, d2)
    d2_ref[...] = d2

    lane_k = jax.lax.broadcasted_iota(jnp.int32, (1, K), 1)

    def body(k, carry):
        vals, idxs = carry
        d2c = d2_ref[...]
        m = jnp.min(d2c, axis=1, keepdims=True)
        cand = jnp.where(d2c == m, col_ids, jnp.int32(2 ** 30))
        sel = jnp.min(cand, axis=1, keepdims=True)
        d2_ref[...] = jnp.where(col_ids == sel, jnp.float32(2e9), d2c)
        vals = jnp.where(lane_k == k, m, vals)
        idxs = jnp.where(lane_k == k, sel, idxs)
        return vals, idxs

    vals0 = jnp.zeros((RB, K), jnp.float32)
    idxs0 = jnp.zeros((RB, K), jnp.int32)
    vals, idxs = jax.lax.fori_loop(0, K, body, (vals0, idxs0))
    d2sel_ref[...] = vals
    eidx_ref[...] = idxs

    xc3 = jnp.concatenate([xr, yr, zr], axis=1)
    xrel = x2 - jnp.concatenate([xc3, xc3, xc3, xc3], axis=1)
    bond = x2[:, 3:12] - x2[:, 0:9]
    lls = []
    for t in range(3):
        b = bond[:, 3 * t:3 * t + 3]
        sq = jnp.sum(b * b, axis=1, keepdims=True)
        lls.append(jnp.log(jnp.sqrt(sq + 1e-8) + 1e-6))
    feat = jnp.concatenate([xrel] + lls, axis=1)
    nh_ref[...] = (jnp.dot(feat, wn_ref[...],
                           preferred_element_type=jnp.float32) + bn_ref[...])
    xc_ref[...] = jnp.concatenate(
        [xc3, jnp.zeros((RB, 125), jnp.float32)], axis=1)


def _knn_embed(x2, x2t, wn, bn):
    return pl.pallas_call(
        _knn_embed_body,
        grid=(NBLK,),
        in_specs=[
            pl.BlockSpec((RB, 12), lambda i: (i, 0)),
            pl.BlockSpec((12, N), lambda i: (0, 0)),
            pl.BlockSpec((15, DH), lambda i: (0, 0)),
            pl.BlockSpec((1, DH), lambda i: (0, 0)),
        ],
        out_specs=[
            pl.BlockSpec((RB, DH), lambda i: (i, 0)),
            pl.BlockSpec((RB, K), lambda i: (i, 0)),
            pl.BlockSpec((RB, K), lambda i: (i, 0)),
            pl.BlockSpec((RB, DH), lambda i: (i, 0)),
        ],
        out_shape=[
            jax.ShapeDtypeStruct((N, DH), jnp.float32),
            jax.ShapeDtypeStruct((N, K), jnp.float32),
            jax.ShapeDtypeStruct((N, K), jnp.int32),
            jax.ShapeDtypeStruct((N, DH), jnp.float32),
        ],
        scratch_shapes=[pltpu.VMEM((RB, N), jnp.float32)],
    )(x2, x2t, wn, bn)


_GW = 128


def _gather_rows(table, idx_pad):
    v = table.shape[1]

    @pl.kernel(
        out_type=jax.ShapeDtypeStruct((EPAD, v), table.dtype),
        mesh=plsc.VectorSubcoreMesh(core_axis_name="core",
                                    subcore_axis_name="subcore"),
    )
    def k(x_hbm, i_hbm, o_hbm):
        def body(i_vmem, o_vmem):
            pltpu.sync_copy(x_hbm.at[i_vmem.at[0]], o_vmem)

        pltpu.emit_pipeline(
            body,
            grid=(EPAD // _GW,),
            in_specs=[pl.BlockSpec((1, _GW), lambda i: (0, i))],
            out_specs=[pl.BlockSpec((_GW, v), lambda i: (i, 0))],
            core_axis_name=("core", "subcore"),
            dimension_semantics=(pltpu.PARALLEL,),
        )(i_hbm, o_hbm)

    return k(table, idx_pad)


def _edge_feat_body(d2_ref, xc_ref, xcj_ref, we_ref, be_ref, eh_ref):
    d2b = d2_ref[...]
    dd = jnp.sqrt(d2b + 1e-8)
    xci = xc_ref[:, 0:3]
    xci = jnp.broadcast_to(xci[:, None, :], (RB, K, 3)).reshape(EB, 3)
    xcj = xcj_ref[...]
    cen = jax.lax.broadcasted_iota(
        jnp.int32, (1, NUM_RBF), 1).astype(jnp.float32) * (20.0 / 15.0)
    sigma = 20.0 / NUM_RBF
    rbf = jnp.exp(-(((dd - cen) / sigma) ** 2))
    dirv = (xcj[:, 0:3] - xci) / (dd + 1e-8)
    ones = jnp.ones((EB, 1), jnp.float32)
    feat = jnp.concatenate([rbf, dirv, ones], axis=1)
    eh_ref[...] = (jnp.dot(feat, we_ref[...],
                           preferred_element_type=jnp.float32) + be_ref[...])


def _edge_feat(d2flat, xc_pad, xcj, we, be):
    return pl.pallas_call(
        _edge_feat_body,
        grid=(NBLK,),
        in_specs=[
            pl.BlockSpec((EB, 1), lambda i: (i, 0)),
            pl.BlockSpec((RB, DH), lambda i: (i, 0)),
            pl.BlockSpec((EB, DH), lambda i: (i, 1)),
            pl.BlockSpec((NUM_RBF + 4, DH), lambda i: (0, 0)),
            pl.BlockSpec((1, DH), lambda i: (0, 0)),
        ],
        out_specs=pl.BlockSpec((EB, DH), lambda i: (i, 0)),
        out_shape=jax.ShapeDtypeStruct((E, DH), jnp.float32),
    )(d2flat, xc_pad, xcj, we, be)


def _msg_node_body(nh_ref, hj_ref, eh_ref, wm1_ref, bm1_ref, wm2_ref, bm2_ref,
                   wd1_ref, bd1_ref, wd2_ref, bd2_ref, out_ref):
    hi = nh_ref[...]
    w1a = wm1_ref[0:DH, :]
    w1b = wm1_ref[DH:2 * DH, :]
    w1c = wm1_ref[2 * DH:3 * DH, :]
    t1 = jnp.dot(hi, w1a, preferred_element_type=jnp.float32) + bm1_ref[...]
    t1r = jnp.broadcast_to(t1[:, None, :], (RB, K, DH)).reshape(EB, DH)
    z = (t1r
         + jnp.dot(hj_ref[...], w1b, preferred_element_type=jnp.float32)
         + jnp.dot(eh_ref[...], w1c, preferred_element_type=jnp.float32))
    m = (jnp.dot(_softplus(z), wm2_ref[...],
                 preferred_element_type=jnp.float32) + bm2_ref[...])
    agg = jnp.sum(m.reshape(RB, K, DH), axis=1) / float(K)
    h = _ln(hi + agg)
    dh = (jnp.dot(_softplus(jnp.dot(h, wd1_ref[...],
                                    preferred_element_type=jnp.float32)
                            + bd1_ref[...]),
                  wd2_ref[...], preferred_element_type=jnp.float32)
          + bd2_ref[...])
    out_ref[...] = _ln(h + dh)


def _msg_node(nh, hj, eh, wm1, bm1, wm2, bm2, wd1, bd1, wd2, bd2):
    return pl.pallas_call(
        _msg_node_body,
        grid=(NBLK,),
        in_specs=[
            pl.BlockSpec((RB, DH), lambda i: (i, 0)),
            pl.BlockSpec((EB, DH), lambda i: (i, 0)),
            pl.BlockSpec((EB, DH), lambda i: (i, 0)),
            pl.BlockSpec((3 * DH, DH), lambda i: (0, 0)),
            pl.BlockSpec((1, DH), lambda i: (0, 0)),
            pl.BlockSpec((DH, DH), lambda i: (0, 0)),
            pl.BlockSpec((1, DH), lambda i: (0, 0)),
            pl.BlockSpec((DH, DH), lambda i: (0, 0)),
            pl.BlockSpec((1, DH), lambda i: (0, 0)),
            pl.BlockSpec((DH, DH), lambda i: (0, 0)),
            pl.BlockSpec((1, DH), lambda i: (0, 0)),
        ],
        out_specs=pl.BlockSpec((RB, DH), lambda i: (i, 0)),
        out_shape=jax.ShapeDtypeStruct((N, DH), jnp.float32),
    )(nh, hj, eh, wm1, bm1, wm2, bm2, wd1, bd1, wd2, bd2)


def _edge_upd_body(nh_ref, hj_ref, eh_ref, we1_ref, be1_ref, we2_ref, be2_ref,
                   out_ref):
    hi = nh_ref[...]
    w1a = we1_ref[0:DH, :]
    w1b = we1_ref[DH:2 * DH, :]
    w1c = we1_ref[2 * DH:3 * DH, :]
    t1 = jnp.dot(hi, w1a, preferred_element_type=jnp.float32) + be1_ref[...]
    t1r = jnp.broadcast_to(t1[:, None, :], (RB, K, DH)).reshape(EB, DH)
    eh = eh_ref[...]
    z = (t1r
         + jnp.dot(hj_ref[...], w1b, preferred_element_type=jnp.float32)
         + jnp.dot(eh, w1c, preferred_element_type=jnp.float32))
    de = (jnp.dot(_softplus(z), we2_ref[...],
                  preferred_element_type=jnp.float32) + be2_ref[...])
    out_ref[...] = _ln(eh + de)


def _edge_upd(nh, hj, eh, we1, be1, we2, be2):
    return pl.pallas_call(
        _edge_upd_body,
        grid=(NBLK,),
        in_specs=[
            pl.BlockSpec((RB, DH), lambda i: (i, 0)),
            pl.BlockSpec((EB, DH), lambda i: (i, 0)),
            pl.BlockSpec((EB, DH), lambda i: (i, 0)),
            pl.BlockSpec((3 * DH, DH), lambda i: (0, 0)),
            pl.BlockSpec((1, DH), lambda i: (0, 0)),
            pl.BlockSpec((DH, DH), lambda i: (0, 0)),
            pl.BlockSpec((1, DH), lambda i: (0, 0)),
        ],
        out_specs=pl.BlockSpec((EB, DH), lambda i: (i, 0)),
        out_shape=jax.ShapeDtypeStruct((E, DH), jnp.float32),
    )(nh, hj, eh, we1, be1, we2, be2)


def kernel(X, C, W_node, b_node, W_edge, b_edge, Wm1, bm1, Wm2, bm2,
           Wd1, bd1, Wd2, bd2, We1, be1, We2, be2):
    B = X.shape[0]
    x2 = X.reshape(N, 12)
    x2t = x2.T
    xc_pad, d2sel, eidx, nh = _knn_embed(
        x2, x2t, W_node, b_node.reshape(1, DH))

    idx_pad = jnp.zeros((1, EPAD), jnp.int32)
    idx_pad = jax.lax.dynamic_update_slice(idx_pad, eidx.reshape(1, E), (0, 0))

    table0 = jnp.concatenate([nh, xc_pad], axis=1)
    g0 = _gather_rows(table0, idx_pad)
    eh = _edge_feat(d2sel.reshape(E, 1), xc_pad, g0,
                    W_edge, b_edge.reshape(1, DH))

    hj = g0
    for l in range(3):
        nh = _msg_node(nh, hj, eh,
                       Wm1[l], bm1[l].reshape(1, DH), Wm2[l],
                       bm2[l].reshape(1, DH), Wd1[l], bd1[l].reshape(1, DH),
                       Wd2[l], bd2[l].reshape(1, DH))
        hj = _gather_rows(nh, idx_pad)
        eh = _edge_upd(nh, hj, eh,
                       We1[l], be1[l].reshape(1, DH), We2[l],
                       be2[l].reshape(1, DH))

    mask_i = (C > 0).astype(jnp.float32)
    mask_ij = jnp.broadcast_to(mask_i[:, :, None], (B, N, K))
    return (nh.reshape(B, N, DH),
            eh.reshape(B, N, K, DH),
            eidx.reshape(B, N, K),
            mask_i,
            mask_ij)


# speedup vs baseline: 10.4471x; 1.3774x over previous
import functools

import jax
import jax.numpy as jnp
from jax.experimental import pallas as pl
from jax.experimental.pallas import tpu as pltpu
from jax.experimental.pallas import tpu_sc as plsc

N = 10000
K = 30
DH = 128
NUM_RBF = 16
E = N * K
EPAD = 307200
RB = 200
EB = RB * K
NBLK = N // RB


_LOG2E = 1.4426950408889634
_LN2 = 0.6931471805599453


def _softplus(x):
    return jnp.maximum(x, 0.0) + _LN2 * jnp.log2(
        1.0 + jnp.exp2(-jnp.abs(x) * _LOG2E))


def _ln(x):
    mu = jnp.mean(x, axis=-1, keepdims=True)
    var = jnp.mean((x - mu) ** 2, axis=-1, keepdims=True)
    return (x - mu) / jnp.sqrt(var + 1e-5)


NP = 10240
NC = NP // 128
DEPTH = 8


def _knn_embed_body(x2_ref, x2tc_ref, wn_ref, bn_ref,
                    xc_ref, d2sel_ref, eidx_ref, nh_ref, vstk_ref, cstk_ref):
    i = pl.program_id(0)
    x2 = x2_ref[...]

    xr = (x2[:, 0:1] + x2[:, 3:4] + x2[:, 6:7] + x2[:, 9:10]) * 0.25
    yr = (x2[:, 1:2] + x2[:, 4:5] + x2[:, 7:8] + x2[:, 10:11]) * 0.25
    zr = (x2[:, 2:3] + x2[:, 5:6] + x2[:, 8:9] + x2[:, 11:12]) * 0.25

    row_ids = i * RB + jax.lax.broadcasted_iota(jnp.int32, (RB, 1), 0)
    lane = jax.lax.broadcasted_iota(jnp.int32, (1, 128), 1)

    for t in range(DEPTH):
        vstk_ref[t] = jnp.full((RB, 128), jnp.inf, jnp.float32)
        cstk_ref[t] = jnp.full((RB, 128), 2 ** 30, jnp.int32)

    def chunk_body(c, _):
        xt2 = x2tc_ref[c]
        xT = (xt2[0:1] + xt2[3:4] + xt2[6:7] + xt2[9:10]) * 0.25
        yT = (xt2[1:2] + xt2[4:5] + xt2[7:8] + xt2[10:11]) * 0.25
        zT = (xt2[2:3] + xt2[5:6] + xt2[8:9] + xt2[11:12]) * 0.25
        dx = xr - xT
        dy = yr - yT
        dz = zr - zT
        d2c = dx * dx + dy * dy + dz * dz
        colnum = c * 128 + lane
        d2c = jnp.where(row_ids == colnum, 1e9, d2c)
        v = d2c
        cc = jnp.broadcast_to(colnum, (RB, 128))
        for t in range(DEPTH):
            vt = vstk_ref[t]
            ct = cstk_ref[t]
            swap = v < vt
            vstk_ref[t] = jnp.where(swap, v, vt)
            cstk_ref[t] = jnp.where(swap, cc, ct)
            v = jnp.where(swap, vt, v)
            cc = jnp.where(swap, ct, cc)
        return 0

    jax.lax.fori_loop(0, NC, chunk_body, 0)

    lane_k = jax.lax.broadcasted_iota(jnp.int32, (1, K), 1)

    def body(k, carry):
        vals, idxs = carry
        vall = vstk_ref[...]
        call = cstk_ref[...]
        m2 = jnp.min(vall, axis=0)
        m = jnp.min(m2, axis=1, keepdims=True)
        cand = jnp.where(vall == m[None], call, jnp.int32(2 ** 30))
        s2 = jnp.min(cand, axis=0)
        sel = jnp.min(s2, axis=1, keepdims=True)
        vstk_ref[...] = jnp.where(call == sel[None], jnp.inf, vall)
        vals = jnp.where(lane_k == k, m, vals)
        idxs = jnp.where(lane_k == k, sel, idxs)
        return vals, idxs

    vals0 = jnp.zeros((RB, K), jnp.float32)
    idxs0 = jnp.zeros((RB, K), jnp.int32)
    vals, idxs = jax.lax.fori_loop(0, K, body, (vals0, idxs0))
    d2sel_ref[...] = vals
    eidx_ref[...] = idxs

    xc3 = jnp.concatenate([xr, yr, zr], axis=1)
    xrel = x2 - jnp.concatenate([xc3, xc3, xc3, xc3], axis=1)
    bond = x2[:, 3:12] - x2[:, 0:9]
    lls = []
    for t in range(3):
        b = bond[:, 3 * t:3 * t + 3]
        sq = jnp.sum(b * b, axis=1, keepdims=True)
        lls.append(jnp.log(jnp.sqrt(sq + 1e-8) + 1e-6))
    feat = jnp.concatenate([xrel] + lls, axis=1)
    nh_ref[...] = (jnp.dot(feat, wn_ref[...],
                           preferred_element_type=jnp.float32) + bn_ref[...])
    xc_ref[...] = jnp.concatenate(
        [xc3, jnp.zeros((RB, 125), jnp.float32)], axis=1)


def _knn_embed(x2, x2tc, wn, bn):
    return pl.pallas_call(
        _knn_embed_body,
        grid=(NBLK,),
        in_specs=[
            pl.BlockSpec((RB, 12), lambda i: (i, 0)),
            pl.BlockSpec((NC, 12, 128), lambda i: (0, 0, 0)),
            pl.BlockSpec((15, DH), lambda i: (0, 0)),
            pl.BlockSpec((1, DH), lambda i: (0, 0)),
        ],
        out_specs=[
            pl.BlockSpec((RB, DH), lambda i: (i, 0)),
            pl.BlockSpec((RB, K), lambda i: (i, 0)),
            pl.BlockSpec((RB, K), lambda i: (i, 0)),
            pl.BlockSpec((RB, DH), lambda i: (i, 0)),
        ],
        out_shape=[
            jax.ShapeDtypeStruct((N, DH), jnp.float32),
            jax.ShapeDtypeStruct((N, K), jnp.float32),
            jax.ShapeDtypeStruct((N, K), jnp.int32),
            jax.ShapeDtypeStruct((N, DH), jnp.float32),
        ],
        scratch_shapes=[pltpu.VMEM((DEPTH, RB, 128), jnp.float32),
                        pltpu.VMEM((DEPTH, RB, 128), jnp.int32)],
    )(x2, x2tc, wn, bn)


_GW = 128


def _gather_rows(table, idx_pad):
    v = table.shape[1]

    @pl.kernel(
        out_type=jax.ShapeDtypeStruct((EPAD, v), table.dtype),
        mesh=plsc.VectorSubcoreMesh(core_axis_name="core",
                                    subcore_axis_name="subcore"),
    )
    def k(x_hbm, i_hbm, o_hbm):
        def body(i_vmem, o_vmem):
            pltpu.sync_copy(x_hbm.at[i_vmem.at[0]], o_vmem)

        pltpu.emit_pipeline(
            body,
            grid=(EPAD // _GW,),
            in_specs=[pl.BlockSpec((1, _GW), lambda i: (0, i))],
            out_specs=[pl.BlockSpec((_GW, v), lambda i: (i, 0))],
            core_axis_name=("core", "subcore"),
            dimension_semantics=(pltpu.PARALLEL,),
        )(i_hbm, o_hbm)

    return k(table, idx_pad)


def _edge_feat_body(d2_ref, xc_ref, xcj_ref, we_ref, be_ref, eh_ref):
    d2b = d2_ref[...]
    dd = jnp.sqrt(d2b + 1e-8)
    xci = xc_ref[:, 0:3]
    xci = jnp.broadcast_to(xci[:, None, :], (RB, K, 3)).reshape(EB, 3)
    xcj = xcj_ref[...]
    cen = jax.lax.broadcasted_iota(
        jnp.int32, (1, NUM_RBF), 1).astype(jnp.float32) * (20.0 / 15.0)
    sigma = 20.0 / NUM_RBF
    rbf = jnp.exp2(-(((dd - cen) / sigma) ** 2) * _LOG2E)
    dirv = (xcj[:, 0:3] - xci) / (dd + 1e-8)
    ones = jnp.ones((EB, 1), jnp.float32)
    feat = jnp.concatenate([rbf, dirv, ones], axis=1)
    eh_ref[...] = (jnp.dot(feat, we_ref[...],
                           preferred_element_type=jnp.float32) + be_ref[...])


def _edge_feat(d2flat, xc_pad, xcj, we, be):
    return pl.pallas_call(
        _edge_feat_body,
        grid=(NBLK,),
        in_specs=[
            pl.BlockSpec((EB, 1), lambda i: (i, 0)),
            pl.BlockSpec((RB, DH), lambda i: (i, 0)),
            pl.BlockSpec((EB, DH), lambda i: (i, 1)),
            pl.BlockSpec((NUM_RBF + 4, DH), lambda i: (0, 0)),
            pl.BlockSpec((1, DH), lambda i: (0, 0)),
        ],
        out_specs=pl.BlockSpec((EB, DH), lambda i: (i, 0)),
        out_shape=jax.ShapeDtypeStruct((E, DH), jnp.float32),
    )(d2flat, xc_pad, xcj, we, be)


def _msg_node_body(nh_ref, hj_ref, eh_ref, wm1_ref, bm1_ref, wm2_ref, bm2_ref,
                   wd1_ref, bd1_ref, wd2_ref, bd2_ref, out_ref):
    hi = nh_ref[...]
    w1a = wm1_ref[0:DH, :]
    w1b = wm1_ref[DH:2 * DH, :]
    w1c = wm1_ref[2 * DH:3 * DH, :]
    t1 = jnp.dot(hi, w1a, preferred_element_type=jnp.float32) + bm1_ref[...]
    t1r = jnp.broadcast_to(t1[:, None, :], (RB, K, DH)).reshape(EB, DH)
    z = (t1r
         + jnp.dot(hj_ref[...], w1b, preferred_element_type=jnp.float32)
         + jnp.dot(eh_ref[...], w1c, preferred_element_type=jnp.float32))
    m = (jnp.dot(_softplus(z), wm2_ref[...],
                 preferred_element_type=jnp.float32) + bm2_ref[...])
    agg = jnp.sum(m.reshape(RB, K, DH), axis=1) / float(K)
    h = _ln(hi + agg)
    dh = (jnp.dot(_softplus(jnp.dot(h, wd1_ref[...],
                                    preferred_element_type=jnp.float32)
                            + bd1_ref[...]),
                  wd2_ref[...], preferred_element_type=jnp.float32)
          + bd2_ref[...])
    out_ref[...] = _ln(h + dh)


def _msg_node(nh, hj, eh, wm1, bm1, wm2, bm2, wd1, bd1, wd2, bd2):
    return pl.pallas_call(
        _msg_node_body,
        grid=(NBLK,),
        in_specs=[
            pl.BlockSpec((RB, DH), lambda i: (i, 0)),
            pl.BlockSpec((EB, DH), lambda i: (i, 0)),
            pl.BlockSpec((EB, DH), lambda i: (i, 0)),
            pl.BlockSpec((3 * DH, DH), lambda i: (0, 0)),
            pl.BlockSpec((1, DH), lambda i: (0, 0)),
            pl.BlockSpec((DH, DH), lambda i: (0, 0)),
            pl.BlockSpec((1, DH), lambda i: (0, 0)),
            pl.BlockSpec((DH, DH), lambda i: (0, 0)),
            pl.BlockSpec((1, DH), lambda i: (0, 0)),
            pl.BlockSpec((DH, DH), lambda i: (0, 0)),
            pl.BlockSpec((1, DH), lambda i: (0, 0)),
        ],
        out_specs=pl.BlockSpec((RB, DH), lambda i: (i, 0)),
        out_shape=jax.ShapeDtypeStruct((N, DH), jnp.float32),
    )(nh, hj, eh, wm1, bm1, wm2, bm2, wd1, bd1, wd2, bd2)


def _edge_upd_body(nh_ref, hj_ref, eh_ref, we1_ref, be1_ref, we2_ref, be2_ref,
                   out_ref):
    hi = nh_ref[...]
    w1a = we1_ref[0:DH, :]
    w1b = we1_ref[DH:2 * DH, :]
    w1c = we1_ref[2 * DH:3 * DH, :]
    t1 = jnp.dot(hi, w1a, preferred_element_type=jnp.float32) + be1_ref[...]
    t1r = jnp.broadcast_to(t1[:, None, :], (RB, K, DH)).reshape(EB, DH)
    eh = eh_ref[...]
    z = (t1r
         + jnp.dot(hj_ref[...], w1b, preferred_element_type=jnp.float32)
         + jnp.dot(eh, w1c, preferred_element_type=jnp.float32))
    de = (jnp.dot(_softplus(z), we2_ref[...],
                  preferred_element_type=jnp.float32) + be2_ref[...])
    out_ref[...] = _ln(eh + de)


def _edge_upd(nh, hj, eh, we1, be1, we2, be2):
    return pl.pallas_call(
        _edge_upd_body,
        grid=(NBLK,),
        in_specs=[
            pl.BlockSpec((RB, DH), lambda i: (i, 0)),
            pl.BlockSpec((EB, DH), lambda i: (i, 0)),
            pl.BlockSpec((EB, DH), lambda i: (i, 0)),
            pl.BlockSpec((3 * DH, DH), lambda i: (0, 0)),
            pl.BlockSpec((1, DH), lambda i: (0, 0)),
            pl.BlockSpec((DH, DH), lambda i: (0, 0)),
            pl.BlockSpec((1, DH), lambda i: (0, 0)),
        ],
        out_specs=pl.BlockSpec((EB, DH), lambda i: (i, 0)),
        out_shape=jax.ShapeDtypeStruct((E, DH), jnp.float32),
    )(nh, hj, eh, we1, be1, we2, be2)


def kernel(X, C, W_node, b_node, W_edge, b_edge, Wm1, bm1, Wm2, bm2,
           Wd1, bd1, Wd2, bd2, We1, be1, We2, be2):
    B = X.shape[0]
    x2 = X.reshape(N, 12)
    x2t_pad = jnp.concatenate(
        [x2.T, jnp.full((12, NP - N), 1e6, jnp.float32)], axis=1)
    x2tc = x2t_pad.reshape(12, NC, 128).transpose(1, 0, 2)
    xc_pad, d2sel, eidx, nh = _knn_embed(
        x2, x2tc, W_node, b_node.reshape(1, DH))

    idx_pad = jnp.zeros((1, EPAD), jnp.int32)
    idx_pad = jax.lax.dynamic_update_slice(idx_pad, eidx.reshape(1, E), (0, 0))

    table0 = jnp.concatenate([nh, xc_pad], axis=1)
    g0 = _gather_rows(table0, idx_pad)
    eh = _edge_feat(d2sel.reshape(E, 1), xc_pad, g0,
                    W_edge, b_edge.reshape(1, DH))

    hj = g0
    for l in range(3):
        nh = _msg_node(nh, hj, eh,
                       Wm1[l], bm1[l].reshape(1, DH), Wm2[l],
                       bm2[l].reshape(1, DH), Wd1[l], bd1[l].reshape(1, DH),
                       Wd2[l], bd2[l].reshape(1, DH))
        hj = _gather_rows(nh, idx_pad)
        eh = _edge_upd(nh, hj, eh,
                       We1[l], be1[l].reshape(1, DH), We2[l],
                       be2[l].reshape(1, DH))

    mask_i = (C > 0).astype(jnp.float32)
    mask_ij = jnp.broadcast_to(mask_i[:, :, None], (B, N, K))
    return (nh.reshape(B, N, DH),
            eh.reshape(B, N, K, DH),
            eidx.reshape(B, N, K),
            mask_i,
            mask_ij)


# speedup vs baseline: 10.4513x; 1.0004x over previous
import functools

import jax
import jax.numpy as jnp
from jax.experimental import pallas as pl
from jax.experimental.pallas import tpu as pltpu
from jax.experimental.pallas import tpu_sc as plsc

N = 10000
K = 30
DH = 128
NUM_RBF = 16
E = N * K
EPAD = 307200
RB = 200
EB = RB * K
NBLK = N // RB


_LOG2E = 1.4426950408889634
_LN2 = 0.6931471805599453


def _softplus(x):
    return jnp.maximum(x, 0.0) + _LN2 * jnp.log2(
        1.0 + jnp.exp2(-jnp.abs(x) * _LOG2E))


def _ln(x):
    mu = jnp.mean(x, axis=-1, keepdims=True)
    var = jnp.mean((x - mu) ** 2, axis=-1, keepdims=True)
    return (x - mu) / jnp.sqrt(var + 1e-5)


NP = 10240
NC = NP // 128
DEPTH = 8


def _knn_embed_body(x2_ref, x2tc_ref, wn_ref, bn_ref,
                    tab_ref, d2sel_ref, eidx_ref, vstk_ref, cstk_ref):
    i = pl.program_id(0)
    x2 = x2_ref[...]

    xr = (x2[:, 0:1] + x2[:, 3:4] + x2[:, 6:7] + x2[:, 9:10]) * 0.25
    yr = (x2[:, 1:2] + x2[:, 4:5] + x2[:, 7:8] + x2[:, 10:11]) * 0.25
    zr = (x2[:, 2:3] + x2[:, 5:6] + x2[:, 8:9] + x2[:, 11:12]) * 0.25

    row_ids = i * RB + jax.lax.broadcasted_iota(jnp.int32, (RB, 1), 0)
    lane = jax.lax.broadcasted_iota(jnp.int32, (1, 128), 1)

    for t in range(DEPTH):
        vstk_ref[t] = jnp.full((RB, 128), jnp.inf, jnp.float32)
        cstk_ref[t] = jnp.full((RB, 128), 2 ** 30, jnp.int32)

    def chunk_body(c, _):
        xt2 = x2tc_ref[c]
        xT = (xt2[0:1] + xt2[3:4] + xt2[6:7] + xt2[9:10]) * 0.25
        yT = (xt2[1:2] + xt2[4:5] + xt2[7:8] + xt2[10:11]) * 0.25
        zT = (xt2[2:3] + xt2[5:6] + xt2[8:9] + xt2[11:12]) * 0.25
        dx = xr - xT
        dy = yr - yT
        dz = zr - zT
        d2c = dx * dx + dy * dy + dz * dz
        colnum = c * 128 + lane
        d2c = jnp.where(row_ids == colnum, 1e9, d2c)
        v = d2c
        cc = jnp.broadcast_to(colnum, (RB, 128))
        for t in range(DEPTH):
            vt = vstk_ref[t]
            ct = cstk_ref[t]
            swap = v < vt
            vstk_ref[t] = jnp.where(swap, v, vt)
            cstk_ref[t] = jnp.where(swap, cc, ct)
            v = jnp.where(swap, vt, v)
            cc = jnp.where(swap, ct, cc)
        return 0

    jax.lax.fori_loop(0, NC, chunk_body, 0)

    lane_k = jax.lax.broadcasted_iota(jnp.int32, (1, K), 1)

    def body(k, carry):
        vals, idxs = carry
        vall = vstk_ref[...]
        call = cstk_ref[...]
        m2 = jnp.min(vall, axis=0)
        m = jnp.min(m2, axis=1, keepdims=True)
        cand = jnp.where(vall == m[None], call, jnp.int32(2 ** 30))
        s2 = jnp.min(cand, axis=0)
        sel = jnp.min(s2, axis=1, keepdims=True)
        vstk_ref[...] = jnp.where(call == sel[None], jnp.inf, vall)
        vals = jnp.where(lane_k == k, m, vals)
        idxs = jnp.where(lane_k == k, sel, idxs)
        return vals, idxs

    vals0 = jnp.zeros((RB, K), jnp.float32)
    idxs0 = jnp.zeros((RB, K), jnp.int32)
    vals, idxs = jax.lax.fori_loop(0, K, body, (vals0, idxs0))
    d2sel_ref[...] = vals
    eidx_ref[...] = idxs

    xc3 = jnp.concatenate([xr, yr, zr], axis=1)
    xrel = x2 - jnp.concatenate([xc3, xc3, xc3, xc3], axis=1)
    bond = x2[:, 3:12] - x2[:, 0:9]
    lls = []
    for t in range(3):
        b = bond[:, 3 * t:3 * t + 3]
        sq = jnp.sum(b * b, axis=1, keepdims=True)
        lls.append(jnp.log(jnp.sqrt(sq + 1e-8) + 1e-6))
    feat = jnp.concatenate([xrel] + lls, axis=1)
    nh = (jnp.dot(feat, wn_ref[...],
                  preferred_element_type=jnp.float32) + bn_ref[...])
    tab_ref[...] = jnp.concatenate(
        [nh, xc3, jnp.zeros((RB, 125), jnp.float32)], axis=1)


def _knn_embed(x2, x2tc, wn, bn):
    return pl.pallas_call(
        _knn_embed_body,
        grid=(NBLK,),
        in_specs=[
            pl.BlockSpec((RB, 12), lambda i: (i, 0)),
            pl.BlockSpec((NC, 12, 128), lambda i: (0, 0, 0)),
            pl.BlockSpec((15, DH), lambda i: (0, 0)),
            pl.BlockSpec((1, DH), lambda i: (0, 0)),
        ],
        out_specs=[
            pl.BlockSpec((RB, 2 * DH), lambda i: (i, 0)),
            pl.BlockSpec((RB, K), lambda i: (i, 0)),
            pl.BlockSpec((RB, K), lambda i: (i, 0)),
        ],
        out_shape=[
            jax.ShapeDtypeStruct((N, 2 * DH), jnp.float32),
            jax.ShapeDtypeStruct((N, K), jnp.float32),
            jax.ShapeDtypeStruct((N, K), jnp.int32),
        ],
        scratch_shapes=[pltpu.VMEM((DEPTH, RB, 128), jnp.float32),
                        pltpu.VMEM((DEPTH, RB, 128), jnp.int32)],
    )(x2, x2tc, wn, bn)


_GW = 128


def _gather_rows(table, idx_pad):
    v = table.shape[1]

    @pl.kernel(
        out_type=jax.ShapeDtypeStruct((EPAD, v), table.dtype),
        mesh=plsc.VectorSubcoreMesh(core_axis_name="core",
                                    subcore_axis_name="subcore"),
    )
    def k(x_hbm, i_hbm, o_hbm):
        def body(i_vmem, o_vmem):
            pltpu.sync_copy(x_hbm.at[i_vmem.at[0]], o_vmem)

        pltpu.emit_pipeline(
            body,
            grid=(EPAD // _GW,),
            in_specs=[pl.BlockSpec((1, _GW), lambda i: (0, i))],
            out_specs=[pl.BlockSpec((_GW, v), lambda i: (i, 0))],
            core_axis_name=("core", "subcore"),
            dimension_semantics=(pltpu.PARALLEL,),
        )(i_hbm, o_hbm)

    return k(table, idx_pad)


def _edge_feat_body(d2_ref, xc_ref, xcj_ref, we_ref, be_ref, eh_ref):
    d2b = d2_ref[...]
    dd = jnp.sqrt(d2b + 1e-8)
    xci = xc_ref[:, 0:3]
    xci = jnp.broadcast_to(xci[:, None, :], (RB, K, 3)).reshape(EB, 3)
    xcj = xcj_ref[...]
    cen = jax.lax.broadcasted_iota(
        jnp.int32, (1, NUM_RBF), 1).astype(jnp.float32) * (20.0 / 15.0)
    sigma = 20.0 / NUM_RBF
    rbf = jnp.exp2(-(((dd - cen) / sigma) ** 2) * _LOG2E)
    dirv = (xcj[:, 0:3] - xci) / (dd + 1e-8)
    ones = jnp.ones((EB, 1), jnp.float32)
    feat = jnp.concatenate([rbf, dirv, ones], axis=1)
    eh_ref[...] = (jnp.dot(feat, we_ref[...],
                           preferred_element_type=jnp.float32) + be_ref[...])


def _edge_feat(d2flat, xc_pad, xcj, we, be):
    return pl.pallas_call(
        _edge_feat_body,
        grid=(NBLK,),
        in_specs=[
            pl.BlockSpec((EB, 1), lambda i: (i, 0)),
            pl.BlockSpec((RB, DH), lambda i: (i, 1)),
            pl.BlockSpec((EB, DH), lambda i: (i, 1)),
            pl.BlockSpec((NUM_RBF + 4, DH), lambda i: (0, 0)),
            pl.BlockSpec((1, DH), lambda i: (0, 0)),
        ],
        out_specs=pl.BlockSpec((EB, DH), lambda i: (i, 0)),
        out_shape=jax.ShapeDtypeStruct((E, DH), jnp.float32),
    )(d2flat, xc_pad, xcj, we, be)


def _msg_node_body(nh_ref, hj_ref, eh_ref, wm1_ref, bm1_ref, wm2_ref, bm2_ref,
                   wd1_ref, bd1_ref, wd2_ref, bd2_ref, out_ref):
    hi = nh_ref[...]
    w1a = wm1_ref[0:DH, :]
    w1b = wm1_ref[DH:2 * DH, :]
    w1c = wm1_ref[2 * DH:3 * DH, :]
    t1 = jnp.dot(hi, w1a, preferred_element_type=jnp.float32) + bm1_ref[...]
    t1r = jnp.broadcast_to(t1[:, None, :], (RB, K, DH)).reshape(EB, DH)
    z = (t1r
         + jnp.dot(hj_ref[...], w1b, preferred_element_type=jnp.float32)
         + jnp.dot(eh_ref[...], w1c, preferred_element_type=jnp.float32))
    m = (jnp.dot(_softplus(z), wm2_ref[...],
                 preferred_element_type=jnp.float32) + bm2_ref[...])
    agg = jnp.sum(m.reshape(RB, K, DH), axis=1) / float(K)
    h = _ln(hi + agg)
    dh = (jnp.dot(_softplus(jnp.dot(h, wd1_ref[...],
                                    preferred_element_type=jnp.float32)
                            + bd1_ref[...]),
                  wd2_ref[...], preferred_element_type=jnp.float32)
          + bd2_ref[...])
    out_ref[...] = _ln(h + dh)


def _msg_node(nh, hj, eh, wm1, bm1, wm2, bm2, wd1, bd1, wd2, bd2):
    return pl.pallas_call(
        _msg_node_body,
        grid=(NBLK,),
        in_specs=[
            pl.BlockSpec((RB, DH), lambda i: (i, 0)),
            pl.BlockSpec((EB, DH), lambda i: (i, 0)),
            pl.BlockSpec((EB, DH), lambda i: (i, 0)),
            pl.BlockSpec((3 * DH, DH), lambda i: (0, 0)),
            pl.BlockSpec((1, DH), lambda i: (0, 0)),
            pl.BlockSpec((DH, DH), lambda i: (0, 0)),
            pl.BlockSpec((1, DH), lambda i: (0, 0)),
            pl.BlockSpec((DH, DH), lambda i: (0, 0)),
            pl.BlockSpec((1, DH), lambda i: (0, 0)),
            pl.BlockSpec((DH, DH), lambda i: (0, 0)),
            pl.BlockSpec((1, DH), lambda i: (0, 0)),
        ],
        out_specs=pl.BlockSpec((RB, DH), lambda i: (i, 0)),
        out_shape=jax.ShapeDtypeStruct((N, DH), jnp.float32),
    )(nh, hj, eh, wm1, bm1, wm2, bm2, wd1, bd1, wd2, bd2)


def _edge_upd_body(nh_ref, hj_ref, eh_ref, we1_ref, be1_ref, we2_ref, be2_ref,
                   out_ref):
    hi = nh_ref[...]
    w1a = we1_ref[0:DH, :]
    w1b = we1_ref[DH:2 * DH, :]
    w1c = we1_ref[2 * DH:3 * DH, :]
    t1 = jnp.dot(hi, w1a, preferred_element_type=jnp.float32) + be1_ref[...]
    t1r = jnp.broadcast_to(t1[:, None, :], (RB, K, DH)).reshape(EB, DH)
    eh = eh_ref[...]
    z = (t1r
         + jnp.dot(hj_ref[...], w1b, preferred_element_type=jnp.float32)
         + jnp.dot(eh, w1c, preferred_element_type=jnp.float32))
    de = (jnp.dot(_softplus(z), we2_ref[...],
                  preferred_element_type=jnp.float32) + be2_ref[...])
    out_ref[...] = _ln(eh + de)


def _edge_upd(nh, hj, eh, we1, be1, we2, be2):
    return pl.pallas_call(
        _edge_upd_body,
        grid=(NBLK,),
        in_specs=[
            pl.BlockSpec((RB, DH), lambda i: (i, 0)),
            pl.BlockSpec((EB, DH), lambda i: (i, 0)),
            pl.BlockSpec((EB, DH), lambda i: (i, 0)),
            pl.BlockSpec((3 * DH, DH), lambda i: (0, 0)),
            pl.BlockSpec((1, DH), lambda i: (0, 0)),
            pl.BlockSpec((DH, DH), lambda i: (0, 0)),
            pl.BlockSpec((1, DH), lambda i: (0, 0)),
        ],
        out_specs=pl.BlockSpec((EB, DH), lambda i: (i, 0)),
        out_shape=jax.ShapeDtypeStruct((E, DH), jnp.float32),
    )(nh, hj, eh, we1, be1, we2, be2)


def kernel(X, C, W_node, b_node, W_edge, b_edge, Wm1, bm1, Wm2, bm2,
           Wd1, bd1, Wd2, bd2, We1, be1, We2, be2):
    B = X.shape[0]
    x2 = X.reshape(N, 12)
    x2t_pad = jnp.concatenate(
        [x2.T, jnp.full((12, NP - N), 1e6, jnp.float32)], axis=1)
    x2tc = x2t_pad.reshape(12, NC, 128).transpose(1, 0, 2)
    table0, d2sel, eidx = _knn_embed(
        x2, x2tc, W_node, b_node.reshape(1, DH))

    idx_pad = jnp.zeros((1, EPAD), jnp.int32)
    idx_pad = jax.lax.dynamic_update_slice(idx_pad, eidx.reshape(1, E), (0, 0))

    g0 = _gather_rows(table0, idx_pad)
    eh = _edge_feat(d2sel.reshape(E, 1), table0, g0,
                    W_edge, b_edge.reshape(1, DH))

    nh = table0
    hj = g0
    for l in range(3):
        nh = _msg_node(nh, hj, eh,
                       Wm1[l], bm1[l].reshape(1, DH), Wm2[l],
                       bm2[l].reshape(1, DH), Wd1[l], bd1[l].reshape(1, DH),
                       Wd2[l], bd2[l].reshape(1, DH))
        hj = _gather_rows(nh, idx_pad)
        eh = _edge_upd(nh, hj, eh,
                       We1[l], be1[l].reshape(1, DH), We2[l],
                       be2[l].reshape(1, DH))

    mask_i = (C > 0).astype(jnp.float32)
    mask_ij = jnp.broadcast_to(mask_i[:, :, None], (B, N, K))
    return (nh.reshape(B, N, DH),
            eh.reshape(B, N, K, DH),
            eidx.reshape(B, N, K),
            mask_i,
            mask_ij)


# speedup vs baseline: 12.5183x; 1.1978x over previous
import functools

import jax
import jax.numpy as jnp
from jax.experimental import pallas as pl
from jax.experimental.pallas import tpu as pltpu
from jax.experimental.pallas import tpu_sc as plsc

N = 10000
K = 30
DH = 128
NUM_RBF = 16
E = N * K
RB = 200
EB = RB * K
NBLK = N // RB
HN = N // 2
HE = HN * K
HBLK = NBLK // 2
HEPAD = 151552


_LOG2E = 1.4426950408889634
_LN2 = 0.6931471805599453


def _softplus(x):
    return jnp.maximum(x, 0.0) + _LN2 * jnp.log2(
        1.0 + jnp.exp2(-jnp.abs(x) * _LOG2E))


def _ln(x):
    mu = jnp.mean(x, axis=-1, keepdims=True)
    var = jnp.mean((x - mu) ** 2, axis=-1, keepdims=True)
    return (x - mu) / jnp.sqrt(var + 1e-5)


NP = 10240
NC = NP // 128
DEPTH = 8


def _knn_embed_body(x2_ref, x2tc_ref, wn_ref, bn_ref,
                    tab_ref, d2sel_ref, eidx_ref, vstk_ref, cstk_ref):
    i = pl.program_id(0)
    x2 = x2_ref[...]

    xr = (x2[:, 0:1] + x2[:, 3:4] + x2[:, 6:7] + x2[:, 9:10]) * 0.25
    yr = (x2[:, 1:2] + x2[:, 4:5] + x2[:, 7:8] + x2[:, 10:11]) * 0.25
    zr = (x2[:, 2:3] + x2[:, 5:6] + x2[:, 8:9] + x2[:, 11:12]) * 0.25

    row_ids = i * RB + jax.lax.broadcasted_iota(jnp.int32, (RB, 1), 0)
    lane = jax.lax.broadcasted_iota(jnp.int32, (1, 128), 1)

    for t in range(DEPTH):
        vstk_ref[t] = jnp.full((RB, 128), jnp.inf, jnp.float32)
        cstk_ref[t] = jnp.full((RB, 128), 2 ** 30, jnp.int32)

    def chunk_body(c, _):
        xt2 = x2tc_ref[c]
        xT = (xt2[0:1] + xt2[3:4] + xt2[6:7] + xt2[9:10]) * 0.25
        yT = (xt2[1:2] + xt2[4:5] + xt2[7:8] + xt2[10:11]) * 0.25
        zT = (xt2[2:3] + xt2[5:6] + xt2[8:9] + xt2[11:12]) * 0.25
        dx = xr - xT
        dy = yr - yT
        dz = zr - zT
        d2c = dx * dx + dy * dy + dz * dz
        colnum = c * 128 + lane
        d2c = jnp.where(row_ids == colnum, 1e9, d2c)
        v = d2c
        cc = jnp.broadcast_to(colnum, (RB, 128))
        for t in range(DEPTH):
            vt = vstk_ref[t]
            ct = cstk_ref[t]
            swap = v < vt
            vstk_ref[t] = jnp.where(swap, v, vt)
            cstk_ref[t] = jnp.where(swap, cc, ct)
            v = jnp.where(swap, vt, v)
            cc = jnp.where(swap, ct, cc)
        return 0

    jax.lax.fori_loop(0, NC, chunk_body, 0)

    lane_k = jax.lax.broadcasted_iota(jnp.int32, (1, K), 1)

    def body(k, carry):
        vals, idxs = carry
        vall = vstk_ref[...]
        call = cstk_ref[...]
        m2 = jnp.min(vall, axis=0)
        m = jnp.min(m2, axis=1, keepdims=True)
        cand = jnp.where(vall == m[None], call, jnp.int32(2 ** 30))
        s2 = jnp.min(cand, axis=0)
        sel = jnp.min(s2, axis=1, keepdims=True)
        vstk_ref[...] = jnp.where(call == sel[None], jnp.inf, vall)
        vals = jnp.where(lane_k == k, m, vals)
        idxs = jnp.where(lane_k == k, sel, idxs)
        return vals, idxs

    vals0 = jnp.zeros((RB, K), jnp.float32)
    idxs0 = jnp.zeros((RB, K), jnp.int32)
    vals, idxs = jax.lax.fori_loop(0, K, body, (vals0, idxs0))
    d2sel_ref[...] = vals
    eidx_ref[...] = idxs

    xc3 = jnp.concatenate([xr, yr, zr], axis=1)
    xrel = x2 - jnp.concatenate([xc3, xc3, xc3, xc3], axis=1)
    bond = x2[:, 3:12] - x2[:, 0:9]
    lls = []
    for t in range(3):
        b = bond[:, 3 * t:3 * t + 3]
        sq = jnp.sum(b * b, axis=1, keepdims=True)
        lls.append(jnp.log(jnp.sqrt(sq + 1e-8) + 1e-6))
    feat = jnp.concatenate([xrel] + lls, axis=1)
    nh = (jnp.dot(feat, wn_ref[...],
                  preferred_element_type=jnp.float32) + bn_ref[...])
    tab_ref[...] = jnp.concatenate(
        [nh, xc3, jnp.zeros((RB, 125), jnp.float32)], axis=1)


def _knn_embed(x2, x2tc, wn, bn):
    return pl.pallas_call(
        _knn_embed_body,
        grid=(NBLK,),
        in_specs=[
            pl.BlockSpec((RB, 12), lambda i: (i, 0)),
            pl.BlockSpec((NC, 12, 128), lambda i: (0, 0, 0)),
            pl.BlockSpec((15, DH), lambda i: (0, 0)),
            pl.BlockSpec((1, DH), lambda i: (0, 0)),
        ],
        out_specs=[
            pl.BlockSpec((RB, 2 * DH), lambda i: (i, 0)),
            pl.BlockSpec((RB, K), lambda i: (i, 0)),
            pl.BlockSpec((RB, K), lambda i: (i, 0)),
        ],
        out_shape=[
            jax.ShapeDtypeStruct((N, 2 * DH), jnp.float32),
            jax.ShapeDtypeStruct((N, K), jnp.float32),
            jax.ShapeDtypeStruct((N, K), jnp.int32),
        ],
        scratch_shapes=[pltpu.VMEM((DEPTH, RB, 128), jnp.float32),
                        pltpu.VMEM((DEPTH, RB, 128), jnp.int32)],
    )(x2, x2tc, wn, bn)


_GW = 128


def _gather_rows(table, idx_pad):
    v = table.shape[1]
    m = idx_pad.shape[1]

    @pl.kernel(
        out_type=jax.ShapeDtypeStruct((m, v), table.dtype),
        mesh=plsc.VectorSubcoreMesh(core_axis_name="core",
                                    subcore_axis_name="subcore"),
    )
    def k(x_hbm, i_hbm, o_hbm):
        def body(i_vmem, o_vmem):
            pltpu.sync_copy(x_hbm.at[i_vmem.at[0]], o_vmem)

        pltpu.emit_pipeline(
            body,
            grid=(m // _GW,),
            in_specs=[pl.BlockSpec((1, _GW), lambda i: (0, i))],
            out_specs=[pl.BlockSpec((_GW, v), lambda i: (i, 0))],
            core_axis_name=("core", "subcore"),
            dimension_semantics=(pltpu.PARALLEL,),
        )(i_hbm, o_hbm)

    return k(table, idx_pad)


def _edge_feat_body(d2_ref, xc_ref, xcj_ref, we_ref, be_ref, eh_ref):
    d2b = d2_ref[...]
    dd = jnp.sqrt(d2b + 1e-8)
    xci = xc_ref[:, 0:3]
    xci = jnp.broadcast_to(xci[:, None, :], (RB, K, 3)).reshape(EB, 3)
    xcj = xcj_ref[...]
    cen = jax.lax.broadcasted_iota(
        jnp.int32, (1, NUM_RBF), 1).astype(jnp.float32) * (20.0 / 15.0)
    sigma = 20.0 / NUM_RBF
    rbf = jnp.exp2(-(((dd - cen) / sigma) ** 2) * _LOG2E)
    dirv = (xcj[:, 0:3] - xci) / (dd + 1e-8)
    ones = jnp.ones((EB, 1), jnp.float32)
    feat = jnp.concatenate([rbf, dirv, ones], axis=1)
    eh_ref[...] = (jnp.dot(feat, we_ref[...],
                           preferred_element_type=jnp.float32) + be_ref[...])


def _edge_feat(d2flat, table0, xcj, we, be, off):
    return pl.pallas_call(
        _edge_feat_body,
        grid=(HBLK,),
        in_specs=[
            pl.BlockSpec((EB, 1), lambda i: (i, 0)),
            pl.BlockSpec((RB, DH), lambda i: (i + off * HBLK, 1)),
            pl.BlockSpec((EB, DH), lambda i: (i, 1)),
            pl.BlockSpec((NUM_RBF + 4, DH), lambda i: (0, 0)),
            pl.BlockSpec((1, DH), lambda i: (0, 0)),
        ],
        out_specs=pl.BlockSpec((EB, DH), lambda i: (i, 0)),
        out_shape=jax.ShapeDtypeStruct((HE, DH), jnp.float32),
    )(d2flat, table0, xcj, we, be)


def _msg_node_body(nh_ref, hj_ref, eh_ref, wm1_ref, bm1_ref, wm2_ref, bm2_ref,
                   wd1_ref, bd1_ref, wd2_ref, bd2_ref, out_ref):
    hi = nh_ref[...]
    w1a = wm1_ref[0:DH, :]
    w1b = wm1_ref[DH:2 * DH, :]
    w1c = wm1_ref[2 * DH:3 * DH, :]
    t1 = jnp.dot(hi, w1a, preferred_element_type=jnp.float32) + bm1_ref[...]
    t1r = jnp.broadcast_to(t1[:, None, :], (RB, K, DH)).reshape(EB, DH)
    z = (t1r
         + jnp.dot(hj_ref[...], w1b, preferred_element_type=jnp.float32)
         + jnp.dot(eh_ref[...], w1c, preferred_element_type=jnp.float32))
    m = (jnp.dot(_softplus(z), wm2_ref[...],
                 preferred_element_type=jnp.float32) + bm2_ref[...])
    agg = jnp.sum(m.reshape(RB, K, DH), axis=1) / float(K)
    h = _ln(hi + agg)
    dh = (jnp.dot(_softplus(jnp.dot(h, wd1_ref[...],
                                    preferred_element_type=jnp.float32)
                            + bd1_ref[...]),
                  wd2_ref[...], preferred_element_type=jnp.float32)
          + bd2_ref[...])
    out_ref[...] = _ln(h + dh)


def _msg_node(nh, hj, eh, wm1, bm1, wm2, bm2, wd1, bd1, wd2, bd2, nh_off):
    return pl.pallas_call(
        _msg_node_body,
        grid=(HBLK,),
        in_specs=[
            pl.BlockSpec((RB, DH), lambda i, o=nh_off: (i + o, 0)),
            pl.BlockSpec((EB, DH), lambda i: (i, 0)),
            pl.BlockSpec((EB, DH), lambda i: (i, 0)),
            pl.BlockSpec((3 * DH, DH), lambda i: (0, 0)),
            pl.BlockSpec((1, DH), lambda i: (0, 0)),
            pl.BlockSpec((DH, DH), lambda i: (0, 0)),
            pl.BlockSpec((1, DH), lambda i: (0, 0)),
            pl.BlockSpec((DH, DH), lambda i: (0, 0)),
            pl.BlockSpec((1, DH), lambda i: (0, 0)),
            pl.BlockSpec((DH, DH), lambda i: (0, 0)),
            pl.BlockSpec((1, DH), lambda i: (0, 0)),
        ],
        out_specs=pl.BlockSpec((RB, DH), lambda i: (i, 0)),
        out_shape=jax.ShapeDtypeStruct((HN, DH), jnp.float32),
    )(nh, hj, eh, wm1, bm1, wm2, bm2, wd1, bd1, wd2, bd2)


def _edge_upd_body(nh_ref, hj_ref, eh_ref, we1_ref, be1_ref, we2_ref, be2_ref,
                   out_ref):
    hi = nh_ref[...]
    w1a = we1_ref[0:DH, :]
    w1b = we1_ref[DH:2 * DH, :]
    w1c = we1_ref[2 * DH:3 * DH, :]
    t1 = jnp.dot(hi, w1a, preferred_element_type=jnp.float32) + be1_ref[...]
    t1r = jnp.broadcast_to(t1[:, None, :], (RB, K, DH)).reshape(EB, DH)
    eh = eh_ref[...]
    z = (t1r
         + jnp.dot(hj_ref[...], w1b, preferred_element_type=jnp.float32)
         + jnp.dot(eh, w1c, preferred_element_type=jnp.float32))
    de = (jnp.dot(_softplus(z), we2_ref[...],
                  preferred_element_type=jnp.float32) + be2_ref[...])
    out_ref[...] = _ln(eh + de)


def _edge_upd(nh, hj, eh, we1, be1, we2, be2):
    return pl.pallas_call(
        _edge_upd_body,
        grid=(HBLK,),
        in_specs=[
            pl.BlockSpec((RB, DH), lambda i: (i, 0)),
            pl.BlockSpec((EB, DH), lambda i: (i, 0)),
            pl.BlockSpec((EB, DH), lambda i: (i, 0)),
            pl.BlockSpec((3 * DH, DH), lambda i: (0, 0)),
            pl.BlockSpec((1, DH), lambda i: (0, 0)),
            pl.BlockSpec((DH, DH), lambda i: (0, 0)),
            pl.BlockSpec((1, DH), lambda i: (0, 0)),
        ],
        out_specs=pl.BlockSpec((EB, DH), lambda i: (i, 0)),
        out_shape=jax.ShapeDtypeStruct((HE, DH), jnp.float32),
    )(nh, hj, eh, we1, be1, we2, be2)


def kernel(X, C, W_node, b_node, W_edge, b_edge, Wm1, bm1, Wm2, bm2,
           Wd1, bd1, Wd2, bd2, We1, be1, We2, be2):
    B = X.shape[0]
    x2 = X.reshape(N, 12)
    x2t_pad = jnp.concatenate(
        [x2.T, jnp.full((12, NP - N), 1e6, jnp.float32)], axis=1)
    x2tc = x2t_pad.reshape(12, NC, 128).transpose(1, 0, 2)
    table0, d2sel, eidx = _knn_embed(
        x2, x2tc, W_node, b_node.reshape(1, DH))

    def pad_idx(half):
        return jnp.concatenate(
            [half.reshape(1, HE), jnp.zeros((1, HEPAD - HE), jnp.int32)],
            axis=1)

    idxA = pad_idx(eidx[:HN])
    idxB = pad_idx(eidx[HN:])

    gA = _gather_rows(table0, idxA)
    gB = _gather_rows(table0, idxB)
    be_ = b_edge.reshape(1, DH)
    ehA = _edge_feat(d2sel[:HN].reshape(HE, 1), table0, gA, W_edge, be_, 0)
    ehB = _edge_feat(d2sel[HN:].reshape(HE, 1), table0, gB, W_edge, be_, 1)

    nhA = nhB = table0
    hjA, hjB = gA, gB
    for l in range(3):
        off = HBLK if nhA is table0 else 0
        wl = (Wm1[l], bm1[l].reshape(1, DH), Wm2[l], bm2[l].reshape(1, DH),
              Wd1[l], bd1[l].reshape(1, DH), Wd2[l], bd2[l].reshape(1, DH))
        nhA = _msg_node(nhA, hjA, ehA, *wl, 0)
        nhB = _msg_node(nhB, hjB, ehB, *wl, off)
        nh_full = jnp.concatenate([nhA, nhB], axis=0)
        hjA = _gather_rows(nh_full, idxA)
        hjB = _gather_rows(nh_full, idxB)
        el = (We1[l], be1[l].reshape(1, DH), We2[l], be2[l].reshape(1, DH))
        ehA = _edge_upd(nhA, hjA, ehA, *el)
        ehB = _edge_upd(nhB, hjB, ehB, *el)

    mask_i = (C > 0).astype(jnp.float32)
    mask_ij = jnp.broadcast_to(mask_i[:, :, None], (B, N, K))
    eh = jnp.concatenate([ehA, ehB], axis=0)
    return (nh_full.reshape(B, N, DH),
            eh.reshape(B, N, K, DH),
            eidx.reshape(B, N, K),
            mask_i,
            mask_ij)
